# Initial kernel scaffold; baseline (speedup 1.0000x reference)
import functools

import jax
import jax.numpy as jnp
from jax import lax
from jax.experimental import pallas as pl
from jax.experimental.pallas import tpu as pltpu
from jax.experimental.pallas import tpu_sc as plsc

_NC = 2
_NS = 16
_ROW = 128


def _sc_aggregate(src2, dst2, tbl, zeros, feature_split):
    n_rows = src2.shape[0]
    n_nodes, d = zeros.shape
    k = 10
    if feature_split:
        r_t = n_rows // _NS
        t_extra = n_rows - _NS * r_t
    else:
        nw = _NC * _NS
        r_t = n_rows // nw
        t_extra = n_rows - nw * r_t
    n_chunks = r_t // k
    leftover = r_t % k
    nz = n_nodes // _NS

    def body(src_hbm, dst_hbm, tbl_hbm, zero_hbm, out_hbm, acc, sidx, didx, rows, sem):
        c = lax.axis_index("c")
        s = lax.axis_index("s")
        zsl = pl.ds(s * nz, nz)
        pltpu.sync_copy(zero_hbm.at[zsl], acc.at[zsl])
        plsc.subcore_barrier()

        if feature_split:
            base = s * r_t
            tview = tbl_hbm.at[c]
            extra_pred = s < t_extra
            extra_row = _NS * r_t + s
        else:
            w = s * _NC + c
            base = w * r_t
            tview = tbl_hbm
            extra_pred = w < t_extra
            extra_row = _NC * _NS * r_t + w

        def do_chunk(r0, kk):
            pltpu.sync_copy(src_hbm.at[pl.ds(r0, kk)], sidx.at[pl.ds(0, kk)])
            pltpu.sync_copy(dst_hbm.at[pl.ds(r0, kk)], didx.at[pl.ds(0, kk)])
            cps = [
                pltpu.async_copy(tview.at[sidx.at[j]],
                                 rows.at[pl.ds(j * _ROW, _ROW)], sem)
                for j in range(kk)
            ]
            for cp in cps:
                cp.wait()
            for j in range(kk):
                pltpu.sync_copy(rows.at[pl.ds(j * _ROW, _ROW)],
                                acc.at[didx.at[j]], add=True)

        def chunk_body(i, carry):
            do_chunk(base + i * k, k)
            return carry

        lax.fori_loop(0, n_chunks, chunk_body, 0)
        if leftover:
            do_chunk(base + n_chunks * k, leftover)

        @pl.when(extra_pred)
        def _():
            do_chunk(extra_row, 1)

        plsc.subcore_barrier()
        pltpu.sync_copy(acc.at[zsl], out_hbm.at[c, zsl])

    kern = pl.kernel(
        body,
        out_type=jax.ShapeDtypeStruct((_NC, n_nodes, d), jnp.float32),
        mesh=plsc.VectorSubcoreMesh(core_axis_name="c", subcore_axis_name="s",
                                    num_cores=_NC, num_subcores=_NS),
        scratch_types=[
            pltpu.VMEM_SHARED((n_nodes, d), jnp.float32),
            pltpu.VMEM((k, _ROW), jnp.int32),
            pltpu.VMEM((k, _ROW), jnp.int32),
            pltpu.VMEM((k * _ROW, d), jnp.float32),
            pltpu.SemaphoreType.DMA,
        ],
    )
    return kern(src2, dst2, tbl, zeros)


_BN = 2000


def _tc_layer1(p1, w1, b1):
    n = p1.shape[1]

    def body(p_ref, w_ref, b_ref, h_ref, inv_ref):
        agg = p_ref[0] + p_ref[1]
        inv = 1.0 / jnp.maximum(agg[:, 4:5], 1.0)
        m = agg[:, 0:4] * inv
        h = jnp.dot(m, w_ref[...], preferred_element_type=jnp.float32) + b_ref[...]
        h = jnp.maximum(h, 0.0)
        h_ref[0] = h[:, 0:16]
        h_ref[1] = h[:, 16:32]
        inv_ref[...] = inv

    return pl.pallas_call(
        body,
        grid=(n // _BN,),
        in_specs=[
            pl.BlockSpec((2, _BN, 8), lambda i: (0, i, 0)),
            pl.BlockSpec((4, 32), lambda i: (0, 0)),
            pl.BlockSpec((1, 32), lambda i: (0, 0)),
        ],
        out_specs=[
            pl.BlockSpec((2, _BN, 16), lambda i: (0, i, 0)),
            pl.BlockSpec((_BN, 1), lambda i: (i, 0)),
        ],
        out_shape=[
            jax.ShapeDtypeStruct((2, n, 16), jnp.float32),
            jax.ShapeDtypeStruct((n, 1), jnp.float32),
        ],
    )(p1, w1, b1)


def _tc_layer2(a2, inv, w2, b2, w3):
    n = a2.shape[1]

    def body(a_ref, inv_ref, w2_ref, b2_ref, w3_ref, z_ref):
        agg = jnp.concatenate([a_ref[0], a_ref[1]], axis=1)
        m = agg * inv_ref[...]
        h = jnp.dot(m, w2_ref[...], preferred_element_type=jnp.float32) + b2_ref[...]
        h = jnp.maximum(h, 0.0)
        z_ref[...] = jnp.dot(h, w3_ref[...], preferred_element_type=jnp.float32)

    return pl.pallas_call(
        body,
        grid=(n // _BN,),
        in_specs=[
            pl.BlockSpec((2, _BN, 16), lambda i: (0, i, 0)),
            pl.BlockSpec((_BN, 1), lambda i: (i, 0)),
            pl.BlockSpec((32, 32), lambda i: (0, 0)),
            pl.BlockSpec((1, 32), lambda i: (0, 0)),
            pl.BlockSpec((32, 1), lambda i: (0, 0)),
        ],
        out_specs=pl.BlockSpec((_BN, 1), lambda i: (i, 0)),
        out_shape=jax.ShapeDtypeStruct((n, 1), jnp.float32),
    )(a2, inv, w2, b2, w3)


def _tc_layer3(q, inv, b3):
    n = q.shape[1]

    def body(q_ref, inv_ref, b_ref, o_ref):
        mz = (q_ref[0] + q_ref[1]) * inv_ref[...]
        o_ref[...] = jnp.maximum(mz + b_ref[...], 0.0)

    return pl.pallas_call(
        body,
        grid=(n // _BN,),
        in_specs=[
            pl.BlockSpec((2, _BN, 1), lambda i: (0, i, 0)),
            pl.BlockSpec((_BN, 1), lambda i: (i, 0)),
            pl.BlockSpec((1, 1), lambda i: (0, 0)),
        ],
        out_specs=pl.BlockSpec((_BN, 1), lambda i: (i, 0)),
        out_shape=jax.ShapeDtypeStruct((n, 1), jnp.float32),
    )(q, inv, b3)


def kernel(x, edge_index, W1, b1, W2, b2, W3, b3):
    n = x.shape[0]
    e = edge_index.shape[1]
    src2 = edge_index[0].reshape(e // _ROW, _ROW)
    dst2 = edge_index[1].reshape(e // _ROW, _ROW)

    t1 = jnp.concatenate(
        [x, jnp.ones((n, 1), jnp.float32), jnp.zeros((n, 3), jnp.float32)], axis=1)
    p1 = _sc_aggregate(src2, dst2, t1, jnp.zeros((n, 8), jnp.float32),
                       feature_split=False)
    h1s, inv = _tc_layer1(p1, W1, b1.reshape(1, -1))
    a2 = _sc_aggregate(src2, dst2, h1s, jnp.zeros((n, 16), jnp.float32),
                       feature_split=True)
    z = _tc_layer2(a2, inv, W2, b2.reshape(1, -1), W3)
    q = _sc_aggregate(src2, dst2, z, jnp.zeros((n, 1), jnp.float32),
                      feature_split=False)
    return _tc_layer3(q, inv, b3.reshape(1, -1))


# speedup vs baseline: 13.8410x; 13.8410x over previous
import functools

import jax
import jax.numpy as jnp
from jax import lax
from jax.experimental import pallas as pl
from jax.experimental.pallas import tpu as pltpu
from jax.experimental.pallas import tpu_sc as plsc

_NC = 2
_NS = 16
_ROW = 128


def _sc_aggregate(src2, dst2, tbl, zeros, feature_split):
    n_rows = src2.shape[0]
    n_nodes, d = zeros.shape
    k = 8
    if feature_split:
        r_t = n_rows // _NS
        assert r_t * _NS == n_rows and r_t % k == 0
    else:
        nw = _NC * _NS
        r_t = n_rows // nw
        assert r_t * nw == n_rows and r_t % k == 0
    n_chunks = r_t // k
    nz = n_nodes // _NS

    def body(src_hbm, dst_hbm, tbl_hbm, zero_hbm, out_hbm, acc, sidx, didx, rows, sem):
        c = lax.axis_index("c")
        s = lax.axis_index("s")
        zsl = pl.ds(s * nz, nz)
        pltpu.sync_copy(zero_hbm.at[zsl], acc.at[zsl])
        plsc.subcore_barrier()

        if feature_split:
            base = s * r_t
            tview = tbl_hbm.at[c]
        else:
            w = s * _NC + c
            base = w * r_t
            tview = tbl_hbm

        def do_chunk(r0, kk):
            pltpu.sync_copy(src_hbm.at[pl.ds(r0, kk)], sidx.at[pl.ds(0, kk)])
            pltpu.sync_copy(dst_hbm.at[pl.ds(r0, kk)], didx.at[pl.ds(0, kk)])
            cps = [
                pltpu.async_copy(tview.at[sidx.at[j]],
                                 rows.at[pl.ds(j * _ROW, _ROW)], sem)
                for j in range(kk)
            ]
            for cp in cps:
                cp.wait()
            for j in range(kk):
                pltpu.sync_copy(rows.at[pl.ds(j * _ROW, _ROW)],
                                acc.at[didx.at[j]], add=True)

        def chunk_body(i, carry):
            do_chunk(base + i * k, k)
            return carry

        lax.fori_loop(0, n_chunks, chunk_body, 0)

        plsc.subcore_barrier()
        pltpu.sync_copy(acc.at[zsl], out_hbm.at[c, zsl])

    kern = pl.kernel(
        body,
        out_type=jax.ShapeDtypeStruct((_NC, n_nodes, d), jnp.float32),
        mesh=plsc.VectorSubcoreMesh(core_axis_name="c", subcore_axis_name="s",
                                    num_cores=_NC, num_subcores=_NS),
        compiler_params=pltpu.CompilerParams(use_tc_tiling_on_sc=False),
        scratch_types=[
            pltpu.VMEM_SHARED((n_nodes, d), jnp.float32),
            pltpu.VMEM((k, _ROW), jnp.int32),
            pltpu.VMEM((k, _ROW), jnp.int32),
            pltpu.VMEM((k * _ROW, d), jnp.float32),
            pltpu.SemaphoreType.DMA,
        ],
    )
    return kern(src2, dst2, tbl, zeros)


_BN = 6272


def _tc_layer1(p1, w1, b1):
    n = p1.shape[1]

    def body(p_ref, w_ref, b_ref, h_ref, inv_ref):
        agg = p_ref[0] + p_ref[1]
        inv = 1.0 / jnp.maximum(agg[:, 4:5], 1.0)
        m = agg[:, 0:4] * inv
        h = jnp.dot(m, w_ref[...], preferred_element_type=jnp.float32) + b_ref[...]
        h = jnp.maximum(h, 0.0)
        h_ref[0] = h[:, 0:16]
        h_ref[1] = h[:, 16:32]
        inv_ref[...] = inv

    return pl.pallas_call(
        body,
        grid=(n // _BN,),
        in_specs=[
            pl.BlockSpec((2, _BN, 8), lambda i: (0, i, 0)),
            pl.BlockSpec((4, 32), lambda i: (0, 0)),
            pl.BlockSpec((1, 32), lambda i: (0, 0)),
        ],
        out_specs=[
            pl.BlockSpec((2, _BN, 16), lambda i: (0, i, 0)),
            pl.BlockSpec((_BN, 1), lambda i: (i, 0)),
        ],
        out_shape=[
            jax.ShapeDtypeStruct((2, n, 16), jnp.float32),
            jax.ShapeDtypeStruct((n, 1), jnp.float32),
        ],
    )(p1, w1, b1)


def _tc_layer2(a2, inv, w2, b2):
    n = a2.shape[1]

    def body(a_ref, inv_ref, w2_ref, b2_ref, h_ref):
        agg = jnp.concatenate([a_ref[0], a_ref[1]], axis=1)
        m = agg * inv_ref[...]
        h = jnp.dot(m, w2_ref[...], preferred_element_type=jnp.float32) + b2_ref[...]
        h = jnp.maximum(h, 0.0)
        h_ref[0] = h[:, 0:16]
        h_ref[1] = h[:, 16:32]

    return pl.pallas_call(
        body,
        grid=(n // _BN,),
        in_specs=[
            pl.BlockSpec((2, _BN, 16), lambda i: (0, i, 0)),
            pl.BlockSpec((_BN, 1), lambda i: (i, 0)),
            pl.BlockSpec((32, 32), lambda i: (0, 0)),
            pl.BlockSpec((1, 32), lambda i: (0, 0)),
        ],
        out_specs=pl.BlockSpec((2, _BN, 16), lambda i: (0, i, 0)),
        out_shape=jax.ShapeDtypeStruct((2, n, 16), jnp.float32),
    )(a2, inv, w2, b2)


def _tc_layer3(a3, inv, w3, b3):
    n = a3.shape[1]

    def body(a_ref, inv_ref, w3_ref, b_ref, o_ref):
        agg = jnp.concatenate([a_ref[0], a_ref[1]], axis=1)
        m = agg * inv_ref[...]
        o = jnp.dot(m, w3_ref[...], preferred_element_type=jnp.float32) + b_ref[...]
        o_ref[...] = jnp.maximum(o, 0.0)

    return pl.pallas_call(
        body,
        grid=(n // _BN,),
        in_specs=[
            pl.BlockSpec((2, _BN, 16), lambda i: (0, i, 0)),
            pl.BlockSpec((_BN, 1), lambda i: (i, 0)),
            pl.BlockSpec((32, 1), lambda i: (0, 0)),
            pl.BlockSpec((1, 1), lambda i: (0, 0)),
        ],
        out_specs=pl.BlockSpec((_BN, 1), lambda i: (i, 0)),
        out_shape=jax.ShapeDtypeStruct((n, 1), jnp.float32),
    )(a3, inv, w3, b3)


def kernel(x, edge_index, W1, b1, W2, b2, W3, b3):
    n = x.shape[0]
    e = edge_index.shape[1]
    assert _BN % (8 * _NS) == 0
    npad = ((n + _BN - 1) // _BN) * _BN
    rows0 = e // _ROW
    rpad = ((rows0 + 255) // 256) * 256
    epad = rpad * _ROW - e
    src_p = jnp.concatenate([edge_index[0], jnp.zeros((epad,), jnp.int32)])
    dst_p = jnp.concatenate([edge_index[1], jnp.full((epad,), n, jnp.int32)])
    src2 = src_p.reshape(rpad, _ROW)
    dst2 = dst_p.reshape(rpad, _ROW)

    t1 = jnp.concatenate(
        [x, jnp.ones((n, 1), jnp.float32), jnp.zeros((n, 3), jnp.float32)], axis=1)
    p1 = _sc_aggregate(src2, dst2, t1, jnp.zeros((npad, 8), jnp.float32),
                       feature_split=False)
    h1s, inv = _tc_layer1(p1, W1, b1.reshape(1, -1))
    a2 = _sc_aggregate(src2, dst2, h1s, jnp.zeros((npad, 16), jnp.float32),
                       feature_split=True)
    h2s = _tc_layer2(a2, inv, W2, b2.reshape(1, -1))
    a3 = _sc_aggregate(src2, dst2, h2s, jnp.zeros((npad, 16), jnp.float32),
                       feature_split=True)
    return _tc_layer3(a3, inv, W3, b3.reshape(1, -1))[:n]


# speedup vs baseline: 15.0745x; 1.0891x over previous
import functools

import jax
import jax.numpy as jnp
from jax import lax
from jax.experimental import pallas as pl
from jax.experimental.pallas import tpu as pltpu
from jax.experimental.pallas import tpu_sc as plsc

_NC = 2
_NS = 16
_ROW = 128


def _sc_aggregate(src2, dst2, tbl, zeros, feature_split):
    n_rows = src2.shape[0]
    n_nodes, d = zeros.shape
    k = 4
    if feature_split:
        r_t = n_rows // _NS
        assert r_t * _NS == n_rows and r_t % k == 0
    else:
        nw = _NC * _NS
        r_t = n_rows // nw
        assert r_t * nw == n_rows and r_t % k == 0
    n_chunks = r_t // k
    n_half = n_chunks // 2
    leftover = n_chunks - 2 * n_half
    nz = n_nodes // _NS

    def body(src_hbm, dst_hbm, tbl_hbm, zero_hbm, out_hbm, acc,
             sidx_a, didx_a, sidx_b, didx_b, rows_a, rows_b, gsem, ssem):
        c = lax.axis_index("c")
        s = lax.axis_index("s")
        zsl = pl.ds(s * nz, nz)
        pltpu.sync_copy(zero_hbm.at[zsl], acc.at[zsl])
        plsc.subcore_barrier()

        if feature_split:
            base = s * r_t
            tview = tbl_hbm.at[c]
        else:
            w = s * _NC + c
            base = w * r_t
            tview = tbl_hbm
        last = base + (n_chunks - 1) * k

        def launch(si, di, rb, r0):
            pltpu.sync_copy(src_hbm.at[pl.ds(r0, k)], si)
            pltpu.sync_copy(dst_hbm.at[pl.ds(r0, k)], di)
            for j in range(k):
                pltpu.async_copy(tview.at[si.at[j]],
                                 rb.at[pl.ds(j * _ROW, _ROW)], gsem)

        def drain_gathers(si, rb):
            for j in range(k):
                pltpu.make_async_copy(tview.at[si.at[j]],
                                      rb.at[pl.ds(j * _ROW, _ROW)], gsem).wait()

        def fire_scatters(di, rb):
            for j in range(k):
                pltpu.async_copy(rb.at[pl.ds(j * _ROW, _ROW)],
                                 acc.at[di.at[j]], ssem, add=True)

        def drain_scatters(di, rb):
            for j in range(k):
                pltpu.make_async_copy(rb.at[pl.ds(j * _ROW, _ROW)],
                                      acc.at[di.at[j]], ssem).wait()

        launch(sidx_a, didx_a, rows_a, base)

        def pair_body(i2, carry):
            a0 = base + (2 * i2) * k
            nxt = jnp.minimum(a0 + 2 * k, last)
            launch(sidx_b, didx_b, rows_b, a0 + k)
            drain_gathers(sidx_a, rows_a)
            fire_scatters(didx_a, rows_a)
            drain_gathers(sidx_b, rows_b)

            @pl.when(i2 > 0)
            def _():
                drain_scatters(didx_b, rows_b)
            drain_scatters(didx_a, rows_a)
            fire_scatters(didx_b, rows_b)
            launch(sidx_a, didx_a, rows_a, nxt)
            return carry

        lax.fori_loop(0, n_half, pair_body, 0)
        drain_gathers(sidx_a, rows_a)
        if leftover:
            fire_scatters(didx_a, rows_a)
            drain_scatters(didx_b, rows_b)
            drain_scatters(didx_a, rows_a)
        else:
            drain_scatters(didx_b, rows_b)

        plsc.subcore_barrier()
        pltpu.sync_copy(acc.at[zsl], out_hbm.at[c, zsl])

    kern = pl.kernel(
        body,
        out_type=jax.ShapeDtypeStruct((_NC, n_nodes, d), jnp.float32),
        mesh=plsc.VectorSubcoreMesh(core_axis_name="c", subcore_axis_name="s",
                                    num_cores=_NC, num_subcores=_NS),
        compiler_params=pltpu.CompilerParams(use_tc_tiling_on_sc=False),
        scratch_types=[
            pltpu.VMEM_SHARED((n_nodes, d), jnp.float32),
            pltpu.VMEM((k, _ROW), jnp.int32),
            pltpu.VMEM((k, _ROW), jnp.int32),
            pltpu.VMEM((k, _ROW), jnp.int32),
            pltpu.VMEM((k, _ROW), jnp.int32),
            pltpu.VMEM((k * _ROW, d), jnp.float32),
            pltpu.VMEM((k * _ROW, d), jnp.float32),
            pltpu.SemaphoreType.DMA,
            pltpu.SemaphoreType.DMA,
        ],
    )
    return kern(src2, dst2, tbl, zeros)


_BN = 6272


def _tc_layer1(p1, w1, b1):
    n = p1.shape[1]

    def body(p_ref, w_ref, b_ref, h_ref, inv_ref):
        agg = p_ref[0] + p_ref[1]
        inv = 1.0 / jnp.maximum(agg[:, 4:5], 1.0)
        m = agg[:, 0:4] * inv
        h = jnp.dot(m, w_ref[...], preferred_element_type=jnp.float32) + b_ref[...]
        h = jnp.maximum(h, 0.0)
        h_ref[0] = h[:, 0:16]
        h_ref[1] = h[:, 16:32]
        inv_ref[...] = inv

    return pl.pallas_call(
        body,
        grid=(n // _BN,),
        in_specs=[
            pl.BlockSpec((2, _BN, 8), lambda i: (0, i, 0)),
            pl.BlockSpec((4, 32), lambda i: (0, 0)),
            pl.BlockSpec((1, 32), lambda i: (0, 0)),
        ],
        out_specs=[
            pl.BlockSpec((2, _BN, 16), lambda i: (0, i, 0)),
            pl.BlockSpec((_BN, 1), lambda i: (i, 0)),
        ],
        out_shape=[
            jax.ShapeDtypeStruct((2, n, 16), jnp.float32),
            jax.ShapeDtypeStruct((n, 1), jnp.float32),
        ],
    )(p1, w1, b1)


def _tc_layer2(a2, inv, w2, b2):
    n = a2.shape[1]

    def body(a_ref, inv_ref, w2_ref, b2_ref, h_ref):
        agg = jnp.concatenate([a_ref[0], a_ref[1]], axis=1)
        m = agg * inv_ref[...]
        h = jnp.dot(m, w2_ref[...], preferred_element_type=jnp.float32) + b2_ref[...]
        h = jnp.maximum(h, 0.0)
        h_ref[0] = h[:, 0:16]
        h_ref[1] = h[:, 16:32]

    return pl.pallas_call(
        body,
        grid=(n // _BN,),
        in_specs=[
            pl.BlockSpec((2, _BN, 16), lambda i: (0, i, 0)),
            pl.BlockSpec((_BN, 1), lambda i: (i, 0)),
            pl.BlockSpec((32, 32), lambda i: (0, 0)),
            pl.BlockSpec((1, 32), lambda i: (0, 0)),
        ],
        out_specs=pl.BlockSpec((2, _BN, 16), lambda i: (0, i, 0)),
        out_shape=jax.ShapeDtypeStruct((2, n, 16), jnp.float32),
    )(a2, inv, w2, b2)


def _tc_layer3(a3, inv, w3, b3):
    n = a3.shape[1]

    def body(a_ref, inv_ref, w3_ref, b_ref, o_ref):
        agg = jnp.concatenate([a_ref[0], a_ref[1]], axis=1)
        m = agg * inv_ref[...]
        o = jnp.dot(m, w3_ref[...], preferred_element_type=jnp.float32) + b_ref[...]
        o_ref[...] = jnp.maximum(o, 0.0)

    return pl.pallas_call(
        body,
        grid=(n // _BN,),
        in_specs=[
            pl.BlockSpec((2, _BN, 16), lambda i: (0, i, 0)),
            pl.BlockSpec((_BN, 1), lambda i: (i, 0)),
            pl.BlockSpec((32, 1), lambda i: (0, 0)),
            pl.BlockSpec((1, 1), lambda i: (0, 0)),
        ],
        out_specs=pl.BlockSpec((_BN, 1), lambda i: (i, 0)),
        out_shape=jax.ShapeDtypeStruct((n, 1), jnp.float32),
    )(a3, inv, w3, b3)


def kernel(x, edge_index, W1, b1, W2, b2, W3, b3):
    n = x.shape[0]
    e = edge_index.shape[1]
    assert _BN % (8 * _NS) == 0
    npad = ((n + _BN - 1) // _BN) * _BN
    rows0 = e // _ROW
    rpad = ((rows0 + 255) // 256) * 256
    epad = rpad * _ROW - e
    src_p = jnp.concatenate([edge_index[0], jnp.zeros((epad,), jnp.int32)])
    dst_p = jnp.concatenate([edge_index[1], jnp.full((epad,), n, jnp.int32)])
    src2 = src_p.reshape(rpad, _ROW)
    dst2 = dst_p.reshape(rpad, _ROW)

    t1 = jnp.concatenate(
        [x, jnp.ones((n, 1), jnp.float32), jnp.zeros((n, 3), jnp.float32)], axis=1)
    p1 = _sc_aggregate(src2, dst2, t1, jnp.zeros((npad, 8), jnp.float32),
                       feature_split=False)
    h1s, inv = _tc_layer1(p1, W1, b1.reshape(1, -1))
    a2 = _sc_aggregate(src2, dst2, h1s, jnp.zeros((npad, 16), jnp.float32),
                       feature_split=True)
    h2s = _tc_layer2(a2, inv, W2, b2.reshape(1, -1))
    a3 = _sc_aggregate(src2, dst2, h2s, jnp.zeros((npad, 16), jnp.float32),
                       feature_split=True)
    return _tc_layer3(a3, inv, W3, b3.reshape(1, -1))[:n]


# speedup vs baseline: 21.8373x; 1.4486x over previous
import functools

import jax
import jax.numpy as jnp
from jax import lax
from jax.experimental import pallas as pl
from jax.experimental.pallas import tpu as pltpu
from jax.experimental.pallas import tpu_sc as plsc

_NC = 2
_NS = 16
_ROW = 128


def _sc_aggregate(src2, dst2, tbl, zeros, feature_split):
    n_rows = src2.shape[0]
    n_nodes, d = zeros.shape
    k = 4
    if feature_split:
        r_t = n_rows // _NS
        assert r_t * _NS == n_rows and r_t % k == 0
    else:
        nw = _NC * _NS
        r_t = n_rows // nw
        assert r_t * nw == n_rows and r_t % k == 0
    n_chunks = r_t // k
    n_half = n_chunks // 2
    leftover = n_chunks - 2 * n_half
    nz = n_nodes // _NS

    def body(src_hbm, dst_hbm, tbl_hbm, zero_hbm, out_hbm, acc,
             sidx_a, didx_a, sidx_b, didx_b, rows_a, rows_b, gsem, ssem):
        c = lax.axis_index("c")
        s = lax.axis_index("s")
        zsl = pl.ds(s * nz, nz)
        pltpu.sync_copy(zero_hbm.at[zsl], acc.at[zsl])
        plsc.subcore_barrier()

        if feature_split:
            base = s * r_t
            tview = tbl_hbm.at[c]
        else:
            w = s * _NC + c
            base = w * r_t
            tview = tbl_hbm
        last = base + (n_chunks - 1) * k

        def launch(si, di, rb, r0):
            pltpu.sync_copy(src_hbm.at[pl.ds(r0, k)], si)
            pltpu.sync_copy(dst_hbm.at[pl.ds(r0, k)], di)
            for j in range(k):
                pltpu.async_copy(tview.at[si.at[j]],
                                 rb.at[pl.ds(j * _ROW, _ROW)], gsem)

        def drain_gathers(si, rb):
            for j in range(k):
                pltpu.make_async_copy(tview.at[si.at[j]],
                                      rb.at[pl.ds(j * _ROW, _ROW)], gsem).wait()

        def fire_scatters(di, rb):
            for j in range(k):
                pltpu.async_copy(rb.at[pl.ds(j * _ROW, _ROW)],
                                 acc.at[di.at[j]], ssem, add=True)

        def drain_scatters(di, rb):
            for j in range(k):
                pltpu.make_async_copy(rb.at[pl.ds(j * _ROW, _ROW)],
                                      acc.at[di.at[j]], ssem).wait()

        launch(sidx_a, didx_a, rows_a, base)

        def pair_body(i2, carry):
            a0 = base + (2 * i2) * k
            nxt = jnp.minimum(a0 + 2 * k, last)
            launch(sidx_b, didx_b, rows_b, a0 + k)
            drain_gathers(sidx_a, rows_a)
            fire_scatters(didx_a, rows_a)
            drain_gathers(sidx_b, rows_b)

            @pl.when(i2 > 0)
            def _():
                drain_scatters(didx_b, rows_b)
            drain_scatters(didx_a, rows_a)
            fire_scatters(didx_b, rows_b)
            launch(sidx_a, didx_a, rows_a, nxt)
            return carry

        lax.fori_loop(0, n_half, pair_body, 0)
        drain_gathers(sidx_a, rows_a)
        if leftover:
            fire_scatters(didx_a, rows_a)
            drain_scatters(didx_b, rows_b)
            drain_scatters(didx_a, rows_a)
        else:
            drain_scatters(didx_b, rows_b)

        plsc.subcore_barrier()
        pltpu.sync_copy(acc.at[zsl], out_hbm.at[c, zsl])

    kern = pl.kernel(
        body,
        out_type=jax.ShapeDtypeStruct((_NC, n_nodes, d), jnp.float32),
        mesh=plsc.VectorSubcoreMesh(core_axis_name="c", subcore_axis_name="s",
                                    num_cores=_NC, num_subcores=_NS),
        compiler_params=pltpu.CompilerParams(use_tc_tiling_on_sc=False),
        scratch_types=[
            pltpu.VMEM_SHARED((n_nodes, d), jnp.float32),
            pltpu.VMEM((k, _ROW), jnp.int32),
            pltpu.VMEM((k, _ROW), jnp.int32),
            pltpu.VMEM((k, _ROW), jnp.int32),
            pltpu.VMEM((k, _ROW), jnp.int32),
            pltpu.VMEM((k * _ROW, d), jnp.float32),
            pltpu.VMEM((k * _ROW, d), jnp.float32),
            pltpu.SemaphoreType.DMA,
            pltpu.SemaphoreType.DMA,
        ],
    )
    return kern(src2, dst2, tbl, zeros)


_BN = 6272
_BF = _BN // 8


def _bc16():
    b = jnp.zeros((16, 16), jnp.float32).at[4, :].set(1.0)
    return jnp.kron(jnp.eye(8, dtype=jnp.float32), b)


def _bd(w16):
    return jnp.kron(jnp.eye(8, dtype=jnp.float32), w16)


def _tc_layer1(p1f, bc, w0, w1, b0, b1):
    f = p1f.shape[1]

    def body(p_ref, bc_ref, w0_ref, w1_ref, b0_ref, b1_ref, h_ref):
        agg = p_ref[0] + p_ref[1]
        invb = jnp.dot(1.0 / jnp.maximum(agg, 1.0), bc_ref[...],
                       preferred_element_type=jnp.float32)
        m = agg * invb
        h_ref[0] = jnp.maximum(
            jnp.dot(m, w0_ref[...], preferred_element_type=jnp.float32)
            + b0_ref[...], 0.0)
        h_ref[1] = jnp.maximum(
            jnp.dot(m, w1_ref[...], preferred_element_type=jnp.float32)
            + b1_ref[...], 0.0)

    full = lambda i: (0, 0)
    return pl.pallas_call(
        body,
        grid=(f // _BF,),
        in_specs=[
            pl.BlockSpec((2, _BF, 128), lambda i: (0, i, 0)),
            pl.BlockSpec((128, 128), full),
            pl.BlockSpec((128, 128), full),
            pl.BlockSpec((128, 128), full),
            pl.BlockSpec((1, 128), full),
            pl.BlockSpec((1, 128), full),
        ],
        out_specs=pl.BlockSpec((2, _BF, 128), lambda i: (0, i, 0)),
        out_shape=jax.ShapeDtypeStruct((2, f, 128), jnp.float32),
    )(p1f, bc, w0, w1, b0, b1)


def _tc_layer2(a2f, p1f, bc, w00, w01, w10, w11, b0, b1):
    f = a2f.shape[1]

    def body(a_ref, p_ref, bc_ref, w00_ref, w01_ref, w10_ref, w11_ref,
             b0_ref, b1_ref, h_ref):
        p = p_ref[0] + p_ref[1]
        invb = jnp.dot(1.0 / jnp.maximum(p, 1.0), bc_ref[...],
                       preferred_element_type=jnp.float32)
        m0 = a_ref[0] * invb
        m1 = a_ref[1] * invb
        h_ref[0] = jnp.maximum(
            jnp.dot(m0, w00_ref[...], preferred_element_type=jnp.float32)
            + jnp.dot(m1, w10_ref[...], preferred_element_type=jnp.float32)
            + b0_ref[...], 0.0)
        h_ref[1] = jnp.maximum(
            jnp.dot(m0, w01_ref[...], preferred_element_type=jnp.float32)
            + jnp.dot(m1, w11_ref[...], preferred_element_type=jnp.float32)
            + b1_ref[...], 0.0)

    full = lambda i: (0, 0)
    return pl.pallas_call(
        body,
        grid=(f // _BF,),
        in_specs=[
            pl.BlockSpec((2, _BF, 128), lambda i: (0, i, 0)),
            pl.BlockSpec((2, _BF, 128), lambda i: (0, i, 0)),
            pl.BlockSpec((128, 128), full),
            pl.BlockSpec((128, 128), full),
            pl.BlockSpec((128, 128), full),
            pl.BlockSpec((128, 128), full),
            pl.BlockSpec((128, 128), full),
            pl.BlockSpec((1, 128), full),
            pl.BlockSpec((1, 128), full),
        ],
        out_specs=pl.BlockSpec((2, _BF, 128), lambda i: (0, i, 0)),
        out_shape=jax.ShapeDtypeStruct((2, f, 128), jnp.float32),
    )(a2f, p1f, bc, w00, w01, w10, w11, b0, b1)


def _tc_layer3(a3f, p1f, bc, w3a, w3b, b3b):
    f = a3f.shape[1]

    def body(a_ref, p_ref, bc_ref, wa_ref, wb_ref, b_ref, o_ref):
        p = p_ref[0] + p_ref[1]
        invb = jnp.dot(1.0 / jnp.maximum(p, 1.0), bc_ref[...],
                       preferred_element_type=jnp.float32)
        o = (jnp.dot(a_ref[0] * invb, wa_ref[...],
                     preferred_element_type=jnp.float32)
             + jnp.dot(a_ref[1] * invb, wb_ref[...],
                       preferred_element_type=jnp.float32)
             + b_ref[...])
        o_ref[...] = jnp.maximum(o, 0.0)

    full = lambda i: (0, 0)
    return pl.pallas_call(
        body,
        grid=(f // _BF,),
        in_specs=[
            pl.BlockSpec((2, _BF, 128), lambda i: (0, i, 0)),
            pl.BlockSpec((2, _BF, 128), lambda i: (0, i, 0)),
            pl.BlockSpec((128, 128), full),
            pl.BlockSpec((128, 8), full),
            pl.BlockSpec((128, 8), full),
            pl.BlockSpec((1, 8), full),
        ],
        out_specs=pl.BlockSpec((_BF, 8), lambda i: (i, 0)),
        out_shape=jax.ShapeDtypeStruct((f, 8), jnp.float32),
    )(a3f, p1f, bc, w3a, w3b, b3b)


def kernel(x, edge_index, W1, b1, W2, b2, W3, b3):
    n = x.shape[0]
    e = edge_index.shape[1]
    assert _BN % (8 * _NS) == 0
    npad = ((n + _BN - 1) // _BN) * _BN
    f = npad // 8
    rows0 = e // _ROW
    rpad = ((rows0 + 255) // 256) * 256
    epad = rpad * _ROW - e
    src_p = jnp.concatenate([edge_index[0], jnp.zeros((epad,), jnp.int32)])
    dst_p = jnp.concatenate([edge_index[1], jnp.full((epad,), n, jnp.int32)])
    src2 = src_p.reshape(rpad, _ROW)
    dst2 = dst_p.reshape(rpad, _ROW)

    bc = _bc16()
    w1e = jnp.pad(W1, ((0, 12), (0, 0)))
    tile8 = lambda v: jnp.tile(v, 8).reshape(1, -1)

    t1 = jnp.concatenate(
        [x, jnp.ones((n, 1), jnp.float32), jnp.zeros((n, 11), jnp.float32)],
        axis=1)
    p1 = _sc_aggregate(src2, dst2, t1, jnp.zeros((npad, 16), jnp.float32),
                       feature_split=False)
    p1f = p1.reshape(2, f, 128)
    h1f = _tc_layer1(p1f, bc, _bd(w1e[:, 0:16]), _bd(w1e[:, 16:32]),
                     tile8(b1[0:16]), tile8(b1[16:32]))
    a2 = _sc_aggregate(src2, dst2, h1f.reshape(2, npad, 16),
                       jnp.zeros((npad, 16), jnp.float32), feature_split=True)
    h2f = _tc_layer2(a2.reshape(2, f, 128), p1f, bc,
                     _bd(W2[0:16, 0:16]), _bd(W2[0:16, 16:32]),
                     _bd(W2[16:32, 0:16]), _bd(W2[16:32, 16:32]),
                     tile8(b2[0:16]), tile8(b2[16:32]))
    a3 = _sc_aggregate(src2, dst2, h2f.reshape(2, npad, 16),
                       jnp.zeros((npad, 16), jnp.float32), feature_split=True)
    outf = _tc_layer3(a3.reshape(2, f, 128), p1f, bc,
                      _bd(W3[0:16]), _bd(W3[16:32]), tile8(b3))
    return outf.reshape(npad, 1)[:n]
